# baseline (speedup 1.0000x reference)
import functools

import jax
import jax.numpy as jnp
import numpy as np
from jax.experimental import pallas as pl
from jax.experimental.pallas import tpu as pltpu

N = 2048
H = 768
DFF = 4 * H
E = 8
NE = E + 1
TB = 512


def _gelu(v):
    return 0.5 * v * (1.0 + jax.lax.erf(v * np.float32(1.0 / np.sqrt(2.0))))


def _moe_body(x_ref, gw_ref, gb_ref, up_ref, upb_ref, dn_ref, dnb_ref,
              out_ref, w_s):
    e = pl.program_id(1)

    @pl.when(e == 0)
    def _router():
        logits = jax.lax.dot_general(
            x_ref[...], gw_ref[...], (((1,), (1,)), ((), ())),
            preferred_element_type=jnp.float32)
        scores = jax.nn.sigmoid(logits + gb_ref[...])
        ids = jax.lax.broadcasted_iota(jnp.int32, (TB, E), 1)
        m1 = jnp.max(scores, axis=1, keepdims=True)
        i1 = jnp.min(jnp.where(scores == m1, ids, E), axis=1, keepdims=True)
        s2 = jnp.where(ids == i1, -jnp.inf, scores)
        m2 = jnp.max(s2, axis=1, keepdims=True)
        i2 = jnp.min(jnp.where(s2 == m2, ids, E), axis=1, keepdims=True)
        denom = m1 + m2 + np.float32(1e-6)
        w = jnp.where(ids == i1, m1, 0.0) + jnp.where(ids == i2, m2, 0.0)
        w_s[...] = w / denom

    ids = jax.lax.broadcasted_iota(jnp.int32, (TB, E), 1)
    wcol = jnp.sum(jnp.where(ids == e, w_s[...], 0.0), axis=1, keepdims=True)
    we = jnp.where(e == E, np.float32(0.1), wcol)

    h = jax.lax.dot_general(
        x_ref[...], up_ref[0], (((1,), (1,)), ((), ())),
        preferred_element_type=jnp.float32)
    h = _gelu(h + upb_ref[0])
    y = jax.lax.dot_general(
        h, dn_ref[0], (((1,), (1,)), ((), ())),
        preferred_element_type=jnp.float32)
    y = y + dnb_ref[0]

    contrib = we * y

    @pl.when(e == 0)
    def _init():
        out_ref[...] = contrib

    @pl.when(e > 0)
    def _acc():
        out_ref[...] += contrib


def kernel(x, gate_W, gate_bias, up_W, up_b, down_W, down_b, sup_W, sup_b,
           sdown_W, sdown_b):
    b, s, h = x.shape
    xf = x.reshape(-1, h)
    up9 = jnp.concatenate([up_W, sup_W[None]], axis=0)
    upb9 = jnp.concatenate([up_b, sup_b[None]], axis=0)[:, None, :]
    dn9 = jnp.concatenate([down_W, sdown_W[None]], axis=0)
    dnb9 = jnp.concatenate([down_b, sdown_b[None]], axis=0)[:, None, :]
    gb = gate_bias[None, :]

    grid = (N // TB, NE)
    out = pl.pallas_call(
        _moe_body,
        grid=grid,
        in_specs=[
            pl.BlockSpec((TB, H), lambda t, e: (t, 0)),
            pl.BlockSpec((E, H), lambda t, e: (0, 0)),
            pl.BlockSpec((1, E), lambda t, e: (0, 0)),
            pl.BlockSpec((1, DFF, H), lambda t, e: (e, 0, 0)),
            pl.BlockSpec((1, 1, DFF), lambda t, e: (e, 0, 0)),
            pl.BlockSpec((1, H, DFF), lambda t, e: (e, 0, 0)),
            pl.BlockSpec((1, 1, H), lambda t, e: (e, 0, 0)),
        ],
        out_specs=pl.BlockSpec((TB, H), lambda t, e: (t, 0)),
        out_shape=jax.ShapeDtypeStruct((N, H), jnp.float32),
        scratch_shapes=[pltpu.VMEM((TB, E), jnp.float32)],
        compiler_params=pltpu.CompilerParams(
            dimension_semantics=("arbitrary", "arbitrary")),
    )(xf, gate_W, gb, up9, upb9, dn9, dnb9)
    return out.reshape(b, s, h)


# speedup vs baseline: 1.0295x; 1.0295x over previous
import functools

import jax
import jax.numpy as jnp
import numpy as np
from jax.experimental import pallas as pl
from jax.experimental.pallas import tpu as pltpu

N = 2048
H = 768
DFF = 4 * H
E = 8
NE = E + 1
TB = 1024


def _gelu(v):
    return 0.5 * v * (1.0 + jax.lax.erf(v * np.float32(1.0 / np.sqrt(2.0))))


def _moe_body(x_ref, gw_ref, gb_ref, up_ref, upb_ref, dn_ref, dnb_ref,
              out_ref, w_s):
    e = pl.program_id(1)

    @pl.when(e == 0)
    def _router():
        logits = jax.lax.dot_general(
            x_ref[...], gw_ref[...], (((1,), (1,)), ((), ())),
            preferred_element_type=jnp.float32)
        scores = jax.nn.sigmoid(logits + gb_ref[...])
        ids = jax.lax.broadcasted_iota(jnp.int32, (TB, E), 1)
        m1 = jnp.max(scores, axis=1, keepdims=True)
        i1 = jnp.min(jnp.where(scores == m1, ids, E), axis=1, keepdims=True)
        s2 = jnp.where(ids == i1, -jnp.inf, scores)
        m2 = jnp.max(s2, axis=1, keepdims=True)
        i2 = jnp.min(jnp.where(s2 == m2, ids, E), axis=1, keepdims=True)
        denom = m1 + m2 + np.float32(1e-6)
        w = jnp.where(ids == i1, m1, 0.0) + jnp.where(ids == i2, m2, 0.0)
        w_s[...] = w / denom

    ids = jax.lax.broadcasted_iota(jnp.int32, (TB, E), 1)
    wcol = jnp.sum(jnp.where(ids == e, w_s[...], 0.0), axis=1, keepdims=True)
    we = jnp.where(e == E, np.float32(0.1), wcol)

    xb = x_ref[...].astype(jnp.bfloat16)
    h = jax.lax.dot_general(
        xb, up_ref[0], (((1,), (1,)), ((), ())),
        preferred_element_type=jnp.float32)
    h = _gelu(h + upb_ref[0]).astype(jnp.bfloat16)
    y = jax.lax.dot_general(
        h, dn_ref[0], (((1,), (1,)), ((), ())),
        preferred_element_type=jnp.float32)
    y = y + dnb_ref[0]

    contrib = we * y

    @pl.when(e == 0)
    def _init():
        out_ref[...] = contrib

    @pl.when(e > 0)
    def _acc():
        out_ref[...] += contrib


def kernel(x, gate_W, gate_bias, up_W, up_b, down_W, down_b, sup_W, sup_b,
           sdown_W, sdown_b):
    b, s, h = x.shape
    xf = x.reshape(-1, h)
    up9 = jnp.concatenate([up_W, sup_W[None]], axis=0).astype(jnp.bfloat16)
    upb9 = jnp.concatenate([up_b, sup_b[None]], axis=0)[:, None, :]
    dn9 = jnp.concatenate([down_W, sdown_W[None]],
                          axis=0).astype(jnp.bfloat16)
    dnb9 = jnp.concatenate([down_b, sdown_b[None]], axis=0)[:, None, :]
    gb = gate_bias[None, :]

    grid = (N // TB, NE)
    out = pl.pallas_call(
        _moe_body,
        grid=grid,
        in_specs=[
            pl.BlockSpec((TB, H), lambda t, e: (t, 0)),
            pl.BlockSpec((E, H), lambda t, e: (0, 0)),
            pl.BlockSpec((1, E), lambda t, e: (0, 0)),
            pl.BlockSpec((1, DFF, H), lambda t, e: (e, 0, 0)),
            pl.BlockSpec((1, 1, DFF), lambda t, e: (e, 0, 0)),
            pl.BlockSpec((1, H, DFF), lambda t, e: (e, 0, 0)),
            pl.BlockSpec((1, 1, H), lambda t, e: (e, 0, 0)),
        ],
        out_specs=pl.BlockSpec((TB, H), lambda t, e: (t, 0)),
        out_shape=jax.ShapeDtypeStruct((N, H), jnp.float32),
        scratch_shapes=[pltpu.VMEM((TB, E), jnp.float32)],
        compiler_params=pltpu.CompilerParams(
            dimension_semantics=("arbitrary", "arbitrary")),
    )(xf, gate_W, gb, up9, upb9, dn9, dnb9)
    return out.reshape(b, s, h)


# speedup vs baseline: 1.3574x; 1.3186x over previous
import functools

import jax
import jax.numpy as jnp
import numpy as np
from jax import lax
from jax.experimental import pallas as pl
from jax.experimental.pallas import tpu as pltpu
from jax.experimental.pallas import tpu_sc as plsc

N = 2048
H = 768
DFF = 4 * H
E = 8
MB = 256
NBLK = 23
P = NBLK * MB
NW = 32
TPW = N // NW


def _gelu(v):
    return 0.5 * v * (1.0 + jax.lax.erf(v * np.float32(1.0 / np.sqrt(2.0))))


def _router_body(x_ref, gw_ref, gb_ref, pos1_ref, pos2_ref, w1_ref, w2_ref,
                 be_ref):
    logits = jax.lax.dot_general(
        x_ref[...], gw_ref[...], (((1,), (1,)), ((), ())),
        preferred_element_type=jnp.float32)
    scores = jax.nn.sigmoid(logits + gb_ref[...])
    ids = jax.lax.broadcasted_iota(jnp.int32, (N, E), 1)
    m1 = jnp.max(scores, axis=1, keepdims=True)
    i1 = jnp.min(jnp.where(scores == m1, ids, E), axis=1, keepdims=True)
    s2 = jnp.where(ids == i1, -jnp.inf, scores)
    m2 = jnp.max(s2, axis=1, keepdims=True)
    i2 = jnp.min(jnp.where(s2 == m2, ids, E), axis=1, keepdims=True)
    denom = m1 + m2 + np.float32(1e-6)
    oh1 = (ids == i1).astype(jnp.float32)
    oh2 = (ids == i2).astype(jnp.float32)

    def cumsum_tokens(v):
        c = v
        s = 1
        while s < N:
            c = c + jnp.concatenate(
                [jnp.zeros((s, E), jnp.float32), c[:-s]], axis=0)
            s *= 2
        return c - v

    r1 = cumsum_tokens(oh1)
    c1 = jnp.sum(oh1, axis=0, keepdims=True)
    r2 = cumsum_tokens(oh2) + c1
    counts = (c1 + jnp.sum(oh2, axis=0, keepdims=True)).astype(jnp.int32)
    blocks = (counts + (MB - 1)) // MB
    bs = blocks
    s = 1
    while s < E:
        bs = bs + jnp.concatenate(
            [jnp.zeros((1, s), jnp.int32), bs[:, :-s]], axis=1)
        s *= 2
    block_start = bs - blocks
    base = (block_start * MB).astype(jnp.float32)
    pos1 = jnp.sum(oh1 * (base + r1), axis=1, keepdims=True)
    pos2 = jnp.sum(oh2 * (base + r2), axis=1, keepdims=True)
    ones16 = jnp.ones((1, 16), jnp.float32)
    ones128 = jnp.ones((1, 128), jnp.float32)
    pos1_ref[...] = (pos1 * ones16).astype(jnp.int32)
    pos2_ref[...] = (pos2 * ones16).astype(jnp.int32)
    w1_ref[...] = (m1 / denom) * ones128
    w2_ref[...] = (m2 / denom) * ones128
    bi = jax.lax.broadcasted_iota(jnp.int32, (32, E), 0)
    be = jnp.sum((bi >= block_start).astype(jnp.int32), axis=1,
                 keepdims=True) - 1
    be = jnp.clip(be, 0, E - 1)
    be_ref[...] = be * jnp.ones((1, 16), jnp.int32)


def _router(xf, gate_W, gb):
    return pl.pallas_call(
        _router_body,
        out_shape=[
            jax.ShapeDtypeStruct((N, 16), jnp.int32),
            jax.ShapeDtypeStruct((N, 16), jnp.int32),
            jax.ShapeDtypeStruct((N, 128), jnp.float32),
            jax.ShapeDtypeStruct((N, 128), jnp.float32),
            jax.ShapeDtypeStruct((32, 16), jnp.int32),
        ],
    )(xf, gate_W, gb)


def _dispatch_body(xf_hbm, p1_hbm, p2_hbm, w1_hbm, w2_hbm,
                   xs_hbm, wr_hbm,
                   p1_v, p2_v, rows_v, w1_v, w2_v, sem):
    wid = lax.axis_index("s") * 2 + lax.axis_index("c")
    b = wid * TPW
    pltpu.sync_copy(p1_hbm.at[pl.ds(b, TPW)], p1_v)
    pltpu.sync_copy(p2_hbm.at[pl.ds(b, TPW)], p2_v)
    pltpu.sync_copy(xf_hbm.at[pl.ds(b, TPW)], rows_v)
    pltpu.sync_copy(w1_hbm.at[pl.ds(b, TPW)], w1_v)
    pltpu.sync_copy(w2_hbm.at[pl.ds(b, TPW)], w2_v)
    c1 = pltpu.async_copy(rows_v, xs_hbm.at[p1_v], sem)
    c2 = pltpu.async_copy(rows_v, xs_hbm.at[p2_v], sem)
    c3 = pltpu.async_copy(w1_v, wr_hbm.at[p1_v], sem)
    c4 = pltpu.async_copy(w2_v, wr_hbm.at[p2_v], sem)
    c1.wait()
    c2.wait()
    c3.wait()
    c4.wait()


def _dispatch(xf, pos1, pos2, w1b, w2b):
    mesh = plsc.VectorSubcoreMesh(core_axis_name="c", subcore_axis_name="s")
    f = functools.partial(
        pl.kernel, mesh=mesh,
        out_type=[
            jax.ShapeDtypeStruct((P, H), jnp.float32),
            jax.ShapeDtypeStruct((P, 128), jnp.float32),
        ],
        scratch_types=[
            pltpu.VMEM((TPW,), jnp.int32),
            pltpu.VMEM((TPW,), jnp.int32),
            pltpu.VMEM((TPW, H), jnp.float32),
            pltpu.VMEM((TPW, 128), jnp.float32),
            pltpu.VMEM((TPW, 128), jnp.float32),
            pltpu.SemaphoreType.DMA,
        ],
    )(_dispatch_body)
    return f(xf, pos1, pos2, w1b, w2b)


def _shared_body(x_ref, up_ref, upb_ref, dn_ref, dnb_ref, o_ref):
    xb = x_ref[...].astype(jnp.bfloat16)
    h = jax.lax.dot_general(
        xb, up_ref[...], (((1,), (1,)), ((), ())),
        preferred_element_type=jnp.float32)
    h = _gelu(h + upb_ref[...]).astype(jnp.bfloat16)
    y = jax.lax.dot_general(
        h, dn_ref[...], (((1,), (1,)), ((), ())),
        preferred_element_type=jnp.float32)
    o_ref[...] = np.float32(0.1) * (y + dnb_ref[...])


def _shared(xf, sup_W, sup_b, sdown_W, sdown_b):
    tb = 1024
    return pl.pallas_call(
        _shared_body,
        grid=(N // tb,),
        in_specs=[
            pl.BlockSpec((tb, H), lambda t: (t, 0)),
            pl.BlockSpec((DFF, H), lambda t: (0, 0)),
            pl.BlockSpec((1, DFF), lambda t: (0, 0)),
            pl.BlockSpec((H, DFF), lambda t: (0, 0)),
            pl.BlockSpec((1, H), lambda t: (0, 0)),
        ],
        out_specs=pl.BlockSpec((tb, H), lambda t: (t, 0)),
        out_shape=jax.ShapeDtypeStruct((N, H), jnp.float32),
        compiler_params=pltpu.CompilerParams(
            dimension_semantics=("arbitrary",)),
    )(xf, sup_W.astype(jnp.bfloat16), sup_b[None, :],
      sdown_W.astype(jnp.bfloat16), sdown_b[None, :])


def _grouped_body(be_ref, xs_ref, wr_ref, up_ref, upb_ref, dn_ref, dnb_ref,
                  ys_ref):
    xb = xs_ref[...].astype(jnp.bfloat16)
    h = jax.lax.dot_general(
        xb, up_ref[0], (((1,), (1,)), ((), ())),
        preferred_element_type=jnp.float32)
    h = _gelu(h + upb_ref[0]).astype(jnp.bfloat16)
    y = jax.lax.dot_general(
        h, dn_ref[0], (((1,), (1,)), ((), ())),
        preferred_element_type=jnp.float32)
    y = y + dnb_ref[0]
    ys_ref[...] = y * wr_ref[:, 0:1]


def _grouped(be, xs, wrow, up_W, up_b, down_W, down_b):
    grid_spec = pltpu.PrefetchScalarGridSpec(
        num_scalar_prefetch=1,
        grid=(NBLK,),
        in_specs=[
            pl.BlockSpec((MB, H), lambda i, be: (i, 0)),
            pl.BlockSpec((MB, 128), lambda i, be: (i, 0)),
            pl.BlockSpec((1, DFF, H), lambda i, be: (be[i], 0, 0)),
            pl.BlockSpec((1, 1, DFF), lambda i, be: (be[i], 0, 0)),
            pl.BlockSpec((1, H, DFF), lambda i, be: (be[i], 0, 0)),
            pl.BlockSpec((1, 1, H), lambda i, be: (be[i], 0, 0)),
        ],
        out_specs=pl.BlockSpec((MB, H), lambda i, be: (i, 0)),
    )
    return pl.pallas_call(
        _grouped_body,
        grid_spec=grid_spec,
        out_shape=jax.ShapeDtypeStruct((P, H), jnp.float32),
        compiler_params=pltpu.CompilerParams(
            dimension_semantics=("arbitrary",)),
    )(be, xs, wrow, up_W.astype(jnp.bfloat16), up_b[:, None, :],
      down_W.astype(jnp.bfloat16), down_b[:, None, :])


def _combine_body(sh_hbm, ys_hbm, p1_hbm, p2_hbm, out_hbm,
                  p1_v, p2_v, acc_v, g_v, sem):
    wid = lax.axis_index("s") * 2 + lax.axis_index("c")
    b = wid * TPW
    pltpu.sync_copy(p1_hbm.at[pl.ds(b, TPW)], p1_v)
    pltpu.sync_copy(p2_hbm.at[pl.ds(b, TPW)], p2_v)
    pltpu.sync_copy(sh_hbm.at[pl.ds(b, TPW)], acc_v)

    nchunk = H // 16

    def add_row(i, _):
        for c in range(nchunk):
            sl = pl.ds(c * 16, 16)
            acc_v[i, sl] = acc_v[i, sl] + g_v[i, sl]
        return 0

    pltpu.async_copy(ys_hbm.at[p1_v], g_v, sem).wait()
    lax.fori_loop(0, TPW, add_row, 0)
    pltpu.async_copy(ys_hbm.at[p2_v], g_v, sem).wait()
    lax.fori_loop(0, TPW, add_row, 0)
    pltpu.sync_copy(acc_v, out_hbm.at[pl.ds(b, TPW)])


def _combine(sh, ys, pos1, pos2):
    mesh = plsc.VectorSubcoreMesh(core_axis_name="c", subcore_axis_name="s")
    f = functools.partial(
        pl.kernel, mesh=mesh,
        out_type=jax.ShapeDtypeStruct((N, H), jnp.float32),
        scratch_types=[
            pltpu.VMEM((TPW,), jnp.int32),
            pltpu.VMEM((TPW,), jnp.int32),
            pltpu.VMEM((TPW, H), jnp.float32),
            pltpu.VMEM((TPW, H), jnp.float32),
            pltpu.SemaphoreType.DMA,
        ],
    )(_combine_body)
    return f(sh, ys, pos1, pos2)


def kernel(x, gate_W, gate_bias, up_W, up_b, down_W, down_b, sup_W, sup_b,
           sdown_W, sdown_b):
    b, s, h = x.shape
    xf = x.reshape(-1, h)
    pos1b, pos2b, w1b, w2b, be2d = _router(xf, gate_W, gate_bias[None, :])
    pos1 = pos1b[:, 0]
    pos2 = pos2b[:, 0]
    be = be2d[:NBLK, 0]
    xs, wrow = _dispatch(xf, pos1, pos2, w1b, w2b)
    sh = _shared(xf, sup_W, sup_b, sdown_W, sdown_b)
    ys = _grouped(be, xs, wrow, up_W, up_b, down_W, down_b)
    out = _combine(sh, ys, pos1, pos2)
    return out.reshape(b, s, h)


# speedup vs baseline: 1.7766x; 1.3088x over previous
import functools

import jax
import jax.numpy as jnp
import numpy as np
from jax import lax
from jax.experimental import pallas as pl
from jax.experimental.pallas import tpu as pltpu
from jax.experimental.pallas import tpu_sc as plsc

N = 2048
H = 768
DFF = 4 * H
E = 8
MB = 256
NBLK = 23
P = NBLK * MB
NW = 32
TPW = N // NW


def _gelu(v):
    return 0.5 * v * (1.0 + jax.lax.erf(v * np.float32(1.0 / np.sqrt(2.0))))


def _router_body(x_ref, gw_ref, gb_ref, pos1_ref, pos2_ref, w1_ref, w2_ref,
                 be_ref):
    logits = jax.lax.dot_general(
        x_ref[...], gw_ref[...], (((1,), (1,)), ((), ())),
        preferred_element_type=jnp.float32)
    scores = jax.nn.sigmoid(logits + gb_ref[...])
    ids = jax.lax.broadcasted_iota(jnp.int32, (N, E), 1)
    m1 = jnp.max(scores, axis=1, keepdims=True)
    i1 = jnp.min(jnp.where(scores == m1, ids, E), axis=1, keepdims=True)
    s2 = jnp.where(ids == i1, -jnp.inf, scores)
    m2 = jnp.max(s2, axis=1, keepdims=True)
    i2 = jnp.min(jnp.where(s2 == m2, ids, E), axis=1, keepdims=True)
    denom = m1 + m2 + np.float32(1e-6)
    oh1 = (ids == i1).astype(jnp.float32)
    oh2 = (ids == i2).astype(jnp.float32)

    def cumsum_tokens(v):
        c = v
        s = 1
        while s < N:
            c = c + jnp.concatenate(
                [jnp.zeros((s, E), jnp.float32), c[:-s]], axis=0)
            s *= 2
        return c - v

    r1 = cumsum_tokens(oh1)
    c1 = jnp.sum(oh1, axis=0, keepdims=True)
    r2 = cumsum_tokens(oh2) + c1
    counts = (c1 + jnp.sum(oh2, axis=0, keepdims=True)).astype(jnp.int32)
    blocks = (counts + (MB - 1)) // MB
    bs = blocks
    s = 1
    while s < E:
        bs = bs + jnp.concatenate(
            [jnp.zeros((1, s), jnp.int32), bs[:, :-s]], axis=1)
        s *= 2
    block_start = bs - blocks
    base = (block_start * MB).astype(jnp.float32)
    pos1 = jnp.sum(oh1 * (base + r1), axis=1, keepdims=True)
    pos2 = jnp.sum(oh2 * (base + r2), axis=1, keepdims=True)
    ones16 = jnp.ones((1, 16), jnp.float32)
    ones128 = jnp.ones((1, 128), jnp.float32)
    pos1_ref[...] = (pos1 * ones16).astype(jnp.int32)
    pos2_ref[...] = (pos2 * ones16).astype(jnp.int32)
    w1_ref[...] = (m1 / denom) * ones128
    w2_ref[...] = (m2 / denom) * ones128
    bi = jax.lax.broadcasted_iota(jnp.int32, (32, E), 0)
    be = jnp.sum((bi >= block_start).astype(jnp.int32), axis=1,
                 keepdims=True) - 1
    be = jnp.clip(be, 0, E - 1)
    be_ref[...] = be * jnp.ones((1, 16), jnp.int32)


def _router(xf, gate_W, gb):
    return pl.pallas_call(
        _router_body,
        out_shape=[
            jax.ShapeDtypeStruct((N, 16), jnp.int32),
            jax.ShapeDtypeStruct((N, 16), jnp.int32),
            jax.ShapeDtypeStruct((N, 128), jnp.float32),
            jax.ShapeDtypeStruct((N, 128), jnp.float32),
            jax.ShapeDtypeStruct((32, 16), jnp.int32),
        ],
    )(xf, gate_W, gb)


def _dispatch_body(xf_hbm, p1_hbm, p2_hbm, w1_hbm, w2_hbm,
                   xs_hbm, wr_hbm,
                   p1_v, p2_v, rows_v, w1_v, w2_v, sem):
    wid = lax.axis_index("s") * 2 + lax.axis_index("c")
    b = wid * TPW
    pltpu.sync_copy(p1_hbm.at[pl.ds(b, TPW)], p1_v)
    pltpu.sync_copy(p2_hbm.at[pl.ds(b, TPW)], p2_v)
    pltpu.sync_copy(xf_hbm.at[pl.ds(b, TPW)], rows_v)
    pltpu.sync_copy(w1_hbm.at[pl.ds(b, TPW)], w1_v)
    pltpu.sync_copy(w2_hbm.at[pl.ds(b, TPW)], w2_v)
    c1 = pltpu.async_copy(rows_v, xs_hbm.at[p1_v], sem)
    c2 = pltpu.async_copy(rows_v, xs_hbm.at[p2_v], sem)
    c3 = pltpu.async_copy(w1_v, wr_hbm.at[p1_v], sem)
    c4 = pltpu.async_copy(w2_v, wr_hbm.at[p2_v], sem)
    c1.wait()
    c2.wait()
    c3.wait()
    c4.wait()


def _dispatch(xf, pos1, pos2, w1b, w2b):
    mesh = plsc.VectorSubcoreMesh(core_axis_name="c", subcore_axis_name="s")
    f = functools.partial(
        pl.kernel, mesh=mesh,
        out_type=[
            jax.ShapeDtypeStruct((P, H), jnp.float32),
            jax.ShapeDtypeStruct((P, 128), jnp.float32),
        ],
        scratch_types=[
            pltpu.VMEM((TPW,), jnp.int32),
            pltpu.VMEM((TPW,), jnp.int32),
            pltpu.VMEM((TPW, H), jnp.float32),
            pltpu.VMEM((TPW, 128), jnp.float32),
            pltpu.VMEM((TPW, 128), jnp.float32),
            pltpu.SemaphoreType.DMA,
        ],
    )(_dispatch_body)
    return f(xf, pos1, pos2, w1b, w2b)


def _shared_body(x_ref, up_ref, upb_ref, dn_ref, dnb_ref, o_ref):
    h = jax.lax.dot_general(
        x_ref[...], up_ref[...], (((1,), (1,)), ((), ())),
        preferred_element_type=jnp.float32)
    h = _gelu(h + upb_ref[...])
    y = jax.lax.dot_general(
        h, dn_ref[...], (((1,), (1,)), ((), ())),
        preferred_element_type=jnp.float32)
    o_ref[...] = np.float32(0.1) * (y + dnb_ref[...])


def _shared(xf, sup_W, sup_b, sdown_W, sdown_b):
    tb = 1024
    return pl.pallas_call(
        _shared_body,
        grid=(N // tb,),
        in_specs=[
            pl.BlockSpec((tb, H), lambda t: (t, 0)),
            pl.BlockSpec((DFF, H), lambda t: (0, 0)),
            pl.BlockSpec((1, DFF), lambda t: (0, 0)),
            pl.BlockSpec((H, DFF), lambda t: (0, 0)),
            pl.BlockSpec((1, H), lambda t: (0, 0)),
        ],
        out_specs=pl.BlockSpec((tb, H), lambda t: (t, 0)),
        out_shape=jax.ShapeDtypeStruct((N, H), jnp.float32),
        compiler_params=pltpu.CompilerParams(
            dimension_semantics=("arbitrary",)),
    )(xf, sup_W, sup_b[None, :], sdown_W, sdown_b[None, :])


def _grouped_body(be_ref, xs_ref, wr_ref, up_ref, upb_ref, dn_ref, dnb_ref,
                  ys_ref):
    h = jax.lax.dot_general(
        xs_ref[...], up_ref[0], (((1,), (1,)), ((), ())),
        preferred_element_type=jnp.float32)
    h = _gelu(h + upb_ref[0])
    y = jax.lax.dot_general(
        h, dn_ref[0], (((1,), (1,)), ((), ())),
        preferred_element_type=jnp.float32)
    y = y + dnb_ref[0]
    ys_ref[...] = y * wr_ref[:, 0:1]


def _grouped(be, xs, wrow, up_W, up_b, down_W, down_b):
    grid_spec = pltpu.PrefetchScalarGridSpec(
        num_scalar_prefetch=1,
        grid=(NBLK,),
        in_specs=[
            pl.BlockSpec((MB, H), lambda i, be: (i, 0)),
            pl.BlockSpec((MB, 128), lambda i, be: (i, 0)),
            pl.BlockSpec((1, DFF, H), lambda i, be: (be[i], 0, 0)),
            pl.BlockSpec((1, 1, DFF), lambda i, be: (be[i], 0, 0)),
            pl.BlockSpec((1, H, DFF), lambda i, be: (be[i], 0, 0)),
            pl.BlockSpec((1, 1, H), lambda i, be: (be[i], 0, 0)),
        ],
        out_specs=pl.BlockSpec((MB, H), lambda i, be: (i, 0)),
    )
    return pl.pallas_call(
        _grouped_body,
        grid_spec=grid_spec,
        out_shape=jax.ShapeDtypeStruct((P, H), jnp.float32),
        compiler_params=pltpu.CompilerParams(
            dimension_semantics=("arbitrary",)),
    )(be, xs, wrow, up_W, up_b[:, None, :], down_W, down_b[:, None, :])


def _combine_body(sh_hbm, ys_hbm, p1_hbm, p2_hbm, out_hbm,
                  p1_v, p2_v, acc_v, g_v, sem):
    wid = lax.axis_index("s") * 2 + lax.axis_index("c")
    b = wid * TPW
    pltpu.sync_copy(p1_hbm.at[pl.ds(b, TPW)], p1_v)
    pltpu.sync_copy(p2_hbm.at[pl.ds(b, TPW)], p2_v)
    pltpu.sync_copy(sh_hbm.at[pl.ds(b, TPW)], acc_v)

    nchunk = H // 16

    def add_row(i, _):
        for c in range(nchunk):
            sl = pl.ds(c * 16, 16)
            acc_v[i, sl] = acc_v[i, sl] + g_v[i, sl]
        return 0

    pltpu.async_copy(ys_hbm.at[p1_v], g_v, sem).wait()
    lax.fori_loop(0, TPW, add_row, 0)
    pltpu.async_copy(ys_hbm.at[p2_v], g_v, sem).wait()
    lax.fori_loop(0, TPW, add_row, 0)
    pltpu.sync_copy(acc_v, out_hbm.at[pl.ds(b, TPW)])


def _combine(sh, ys, pos1, pos2):
    mesh = plsc.VectorSubcoreMesh(core_axis_name="c", subcore_axis_name="s")
    f = functools.partial(
        pl.kernel, mesh=mesh,
        out_type=jax.ShapeDtypeStruct((N, H), jnp.float32),
        scratch_types=[
            pltpu.VMEM((TPW,), jnp.int32),
            pltpu.VMEM((TPW,), jnp.int32),
            pltpu.VMEM((TPW, H), jnp.float32),
            pltpu.VMEM((TPW, H), jnp.float32),
            pltpu.SemaphoreType.DMA,
        ],
    )(_combine_body)
    return f(sh, ys, pos1, pos2)


def kernel(x, gate_W, gate_bias, up_W, up_b, down_W, down_b, sup_W, sup_b,
           sdown_W, sdown_b):
    b, s, h = x.shape
    xf = x.reshape(-1, h)
    pos1b, pos2b, w1b, w2b, be2d = _router(xf, gate_W, gate_bias[None, :])
    pos1 = pos1b[:, 0]
    pos2 = pos2b[:, 0]
    be = be2d[:NBLK, 0]
    xs, wrow = _dispatch(xf, pos1, pos2, w1b, w2b)
    sh = _shared(xf, sup_W, sup_b, sdown_W, sdown_b)
    ys = _grouped(be, xs, wrow, up_W, up_b, down_W, down_b)
    out = _combine(sh, ys, pos1, pos2)
    return out.reshape(b, s, h)


# speedup vs baseline: 1.8101x; 1.0188x over previous
import functools

import jax
import jax.numpy as jnp
import numpy as np
from jax import lax
from jax.experimental import pallas as pl
from jax.experimental.pallas import tpu as pltpu
from jax.experimental.pallas import tpu_sc as plsc

N = 2048
H = 768
DFF = 4 * H
E = 8
MB = 256
NBLK = 23
P = NBLK * MB
NW = 32
TPW = N // NW
DC = DFF // 2


def _gelu(v):
    return 0.5 * v * (1.0 + jax.lax.erf(v * np.float32(1.0 / np.sqrt(2.0))))


def _router_body(x_ref, gw_ref, gb_ref, sup_ref, supb_ref, sdn_ref, sdnb_ref,
                 pos1_ref, pos2_ref, w1_ref, w2_ref, be_ref, act_ref, sh_ref):
    logits = jax.lax.dot_general(
        x_ref[...], gw_ref[...], (((1,), (1,)), ((), ())),
        preferred_element_type=jnp.float32)
    scores = jax.nn.sigmoid(logits + gb_ref[...])
    ids = jax.lax.broadcasted_iota(jnp.int32, (N, E), 1)
    m1 = jnp.max(scores, axis=1, keepdims=True)
    i1 = jnp.min(jnp.where(scores == m1, ids, E), axis=1, keepdims=True)
    s2 = jnp.where(ids == i1, -jnp.inf, scores)
    m2 = jnp.max(s2, axis=1, keepdims=True)
    i2 = jnp.min(jnp.where(s2 == m2, ids, E), axis=1, keepdims=True)
    denom = m1 + m2 + np.float32(1e-6)
    oh1 = (ids == i1).astype(jnp.float32)
    oh2 = (ids == i2).astype(jnp.float32)

    def cumsum_tokens(v):
        c = v
        s = 1
        while s < N:
            c = c + jnp.concatenate(
                [jnp.zeros((s, E), jnp.float32), c[:-s]], axis=0)
            s *= 2
        return c - v

    r1 = cumsum_tokens(oh1)
    c1 = jnp.sum(oh1, axis=0, keepdims=True)
    r2 = cumsum_tokens(oh2) + c1
    counts = (c1 + jnp.sum(oh2, axis=0, keepdims=True)).astype(jnp.int32)
    blocks = (counts + (MB - 1)) // MB
    bs = blocks
    s = 1
    while s < E:
        bs = bs + jnp.concatenate(
            [jnp.zeros((1, s), jnp.int32), bs[:, :-s]], axis=1)
        s *= 2
    block_start = bs - blocks
    base = (block_start * MB).astype(jnp.float32)
    pos1 = jnp.sum(oh1 * (base + r1), axis=1, keepdims=True)
    pos2 = jnp.sum(oh2 * (base + r2), axis=1, keepdims=True)
    ones16 = jnp.ones((1, 16), jnp.float32)
    ones128 = jnp.ones((1, 128), jnp.float32)
    pos1_ref[...] = (pos1 * ones16).astype(jnp.int32)
    pos2_ref[...] = (pos2 * ones16).astype(jnp.int32)
    w1_ref[...] = (m1 / denom) * ones128
    w2_ref[...] = (m2 / denom) * ones128
    bi = jax.lax.broadcasted_iota(jnp.int32, (32, E), 0)
    be = jnp.sum((bi >= block_start).astype(jnp.int32), axis=1,
                 keepdims=True) - 1
    be = jnp.clip(be, 0, E - 1)
    be_ref[...] = be * jnp.ones((1, 16), jnp.int32)
    total = jnp.sum(blocks, axis=1, keepdims=True)
    bi16 = jax.lax.broadcasted_iota(jnp.int32, (32, 16), 0)
    act_ref[...] = (bi16 < total).astype(jnp.int32)

    y = jnp.zeros((N, H), jnp.float32) + sdnb_ref[...]
    for k in range(DFF // DC):
        hk = jax.lax.dot_general(
            x_ref[...], sup_ref[pl.ds(k * DC, DC), :],
            (((1,), (1,)), ((), ())), preferred_element_type=jnp.float32)
        hk = _gelu(hk + supb_ref[:, pl.ds(k * DC, DC)])
        y = y + jax.lax.dot_general(
            hk, sdn_ref[:, pl.ds(k * DC, DC)],
            (((1,), (1,)), ((), ())), preferred_element_type=jnp.float32)
    sh_ref[...] = np.float32(0.1) * y


def _router(xf, gate_W, gb, sup_W, sup_b, sdown_W, sdown_b):
    return pl.pallas_call(
        _router_body,
        out_shape=[
            jax.ShapeDtypeStruct((N, 16), jnp.int32),
            jax.ShapeDtypeStruct((N, 16), jnp.int32),
            jax.ShapeDtypeStruct((N, 128), jnp.float32),
            jax.ShapeDtypeStruct((N, 128), jnp.float32),
            jax.ShapeDtypeStruct((32, 16), jnp.int32),
            jax.ShapeDtypeStruct((32, 16), jnp.int32),
            jax.ShapeDtypeStruct((N, H), jnp.float32),
        ],
    )(xf, gate_W, gb, sup_W, sup_b[None, :], sdown_W, sdown_b[None, :])


def _dispatch_body(xf_hbm, p1_hbm, p2_hbm, xs_hbm, p1_v, p2_v, rows_v, sem):
    wid = lax.axis_index("s") * 2 + lax.axis_index("c")
    b = wid * TPW
    pltpu.sync_copy(p1_hbm.at[pl.ds(b, TPW)], p1_v)
    pltpu.sync_copy(p2_hbm.at[pl.ds(b, TPW)], p2_v)
    pltpu.sync_copy(xf_hbm.at[pl.ds(b, TPW)], rows_v)
    c1 = pltpu.async_copy(rows_v, xs_hbm.at[p1_v], sem)
    c2 = pltpu.async_copy(rows_v, xs_hbm.at[p2_v], sem)
    c1.wait()
    c2.wait()


def _dispatch(xf, pos1, pos2):
    mesh = plsc.VectorSubcoreMesh(core_axis_name="c", subcore_axis_name="s")
    f = functools.partial(
        pl.kernel, mesh=mesh,
        out_type=jax.ShapeDtypeStruct((P, H), jnp.float32),
        scratch_types=[
            pltpu.VMEM((TPW,), jnp.int32),
            pltpu.VMEM((TPW,), jnp.int32),
            pltpu.VMEM((TPW, H), jnp.float32),
            pltpu.SemaphoreType.DMA,
        ],
    )(_dispatch_body)
    return f(xf, pos1, pos2)


def _grouped_body(be_ref, act_ref, xs_ref, up_ref, upb_ref, dn_ref, dnb_ref,
                  ys_ref):
    i = pl.program_id(0)

    @pl.when(act_ref[i] != 0)
    def _compute():
        h = jax.lax.dot_general(
            xs_ref[...], up_ref[0], (((1,), (1,)), ((), ())),
            preferred_element_type=jnp.float32)
        h = _gelu(h + upb_ref[0])
        y = jax.lax.dot_general(
            h, dn_ref[0], (((1,), (1,)), ((), ())),
            preferred_element_type=jnp.float32)
        ys_ref[...] = y + dnb_ref[0]


def _grouped(be, act, xs, up_W, up_b, down_W, down_b):
    grid_spec = pltpu.PrefetchScalarGridSpec(
        num_scalar_prefetch=2,
        grid=(NBLK,),
        in_specs=[
            pl.BlockSpec((MB, H), lambda i, be, act: (i, 0)),
            pl.BlockSpec((1, DFF, H), lambda i, be, act: (be[i], 0, 0)),
            pl.BlockSpec((1, 1, DFF), lambda i, be, act: (be[i], 0, 0)),
            pl.BlockSpec((1, H, DFF), lambda i, be, act: (be[i], 0, 0)),
            pl.BlockSpec((1, 1, H), lambda i, be, act: (be[i], 0, 0)),
        ],
        out_specs=pl.BlockSpec((MB, H), lambda i, be, act: (i, 0)),
    )
    return pl.pallas_call(
        _grouped_body,
        grid_spec=grid_spec,
        out_shape=jax.ShapeDtypeStruct((P, H), jnp.float32),
        compiler_params=pltpu.CompilerParams(
            dimension_semantics=("arbitrary",)),
    )(be, act, xs, up_W, up_b[:, None, :], down_W, down_b[:, None, :])


def _combine_body(sh_hbm, ys_hbm, p1_hbm, p2_hbm, w1_hbm, w2_hbm, out_hbm,
                  p1_v, p2_v, acc_v, g_v, w_v, sem):
    wid = lax.axis_index("s") * 2 + lax.axis_index("c")
    b = wid * TPW
    pltpu.sync_copy(p1_hbm.at[pl.ds(b, TPW)], p1_v)
    pltpu.sync_copy(p2_hbm.at[pl.ds(b, TPW)], p2_v)
    pltpu.sync_copy(sh_hbm.at[pl.ds(b, TPW)], acc_v)

    nchunk = H // 16

    def add_row(i, _):
        wv = w_v[i, pl.ds(0, 16)]
        for c in range(nchunk):
            sl = pl.ds(c * 16, 16)
            acc_v[i, sl] = acc_v[i, sl] + wv * g_v[i, sl]
        return 0

    pltpu.sync_copy(w1_hbm.at[pl.ds(b, TPW)], w_v)
    pltpu.async_copy(ys_hbm.at[p1_v], g_v, sem).wait()
    lax.fori_loop(0, TPW, add_row, 0)
    pltpu.sync_copy(w2_hbm.at[pl.ds(b, TPW)], w_v)
    pltpu.async_copy(ys_hbm.at[p2_v], g_v, sem).wait()
    lax.fori_loop(0, TPW, add_row, 0)
    pltpu.sync_copy(acc_v, out_hbm.at[pl.ds(b, TPW)])


def _combine(sh, ys, pos1, pos2, w1b, w2b):
    mesh = plsc.VectorSubcoreMesh(core_axis_name="c", subcore_axis_name="s")
    f = functools.partial(
        pl.kernel, mesh=mesh,
        out_type=jax.ShapeDtypeStruct((N, H), jnp.float32),
        scratch_types=[
            pltpu.VMEM((TPW,), jnp.int32),
            pltpu.VMEM((TPW,), jnp.int32),
            pltpu.VMEM((TPW, H), jnp.float32),
            pltpu.VMEM((TPW, H), jnp.float32),
            pltpu.VMEM((TPW, 128), jnp.float32),
            pltpu.SemaphoreType.DMA,
        ],
    )(_combine_body)
    return f(sh, ys, pos1, pos2, w1b, w2b)


def kernel(x, gate_W, gate_bias, up_W, up_b, down_W, down_b, sup_W, sup_b,
           sdown_W, sdown_b):
    b, s, h = x.shape
    xf = x.reshape(-1, h)
    pos1b, pos2b, w1b, w2b, be2d, act2d, sh = _router(
        xf, gate_W, gate_bias[None, :], sup_W, sup_b, sdown_W, sdown_b)
    pos1 = pos1b[:, 0]
    pos2 = pos2b[:, 0]
    be = be2d[:NBLK, 0]
    act = act2d[:NBLK, 0]
    xs = _dispatch(xf, pos1, pos2)
    ys = _grouped(be, act, xs, up_W, up_b, down_W, down_b)
    out = _combine(sh, ys, pos1, pos2, w1b, w2b)
    return out.reshape(b, s, h)


# speedup vs baseline: 1.8288x; 1.0103x over previous
import functools

import jax
import jax.numpy as jnp
import numpy as np
from jax import lax
from jax.experimental import pallas as pl
from jax.experimental.pallas import tpu as pltpu
from jax.experimental.pallas import tpu_sc as plsc

N = 2048
H = 768
DFF = 4 * H
E = 8
MB = 256
NBLK = 23
P = NBLK * MB
NW = 32
TPW = N // NW
DC = DFF // 2


def _gelu(v):
    return 0.5 * v * (1.0 + jax.lax.erf(v * np.float32(1.0 / np.sqrt(2.0))))


def _router_body(x_ref, gw_ref, gb_ref, sup_ref, supb_ref, sdn_ref, sdnb_ref,
                 pos1_ref, pos2_ref, w1_ref, w2_ref, be_ref, act_ref, sh_ref):
    logits = jax.lax.dot_general(
        x_ref[...], gw_ref[...], (((1,), (1,)), ((), ())),
        preferred_element_type=jnp.float32)
    scores = jax.nn.sigmoid(logits + gb_ref[...])
    ids = jax.lax.broadcasted_iota(jnp.int32, (N, E), 1)
    m1 = jnp.max(scores, axis=1, keepdims=True)
    i1 = jnp.min(jnp.where(scores == m1, ids, E), axis=1, keepdims=True)
    s2 = jnp.where(ids == i1, -jnp.inf, scores)
    m2 = jnp.max(s2, axis=1, keepdims=True)
    i2 = jnp.min(jnp.where(s2 == m2, ids, E), axis=1, keepdims=True)
    denom = m1 + m2 + np.float32(1e-6)
    oh1 = (ids == i1).astype(jnp.float32)
    oh2 = (ids == i2).astype(jnp.float32)

    def cumsum_tokens(v):
        c = v
        s = 1
        while s < N:
            c = c + jnp.concatenate(
                [jnp.zeros((s, E), jnp.float32), c[:-s]], axis=0)
            s *= 2
        return c - v

    r1 = cumsum_tokens(oh1)
    c1 = jnp.sum(oh1, axis=0, keepdims=True)
    r2 = cumsum_tokens(oh2) + c1
    counts = (c1 + jnp.sum(oh2, axis=0, keepdims=True)).astype(jnp.int32)
    blocks = (counts + (MB - 1)) // MB
    bs = blocks
    s = 1
    while s < E:
        bs = bs + jnp.concatenate(
            [jnp.zeros((1, s), jnp.int32), bs[:, :-s]], axis=1)
        s *= 2
    block_start = bs - blocks
    base = (block_start * MB).astype(jnp.float32)
    pos1 = jnp.sum(oh1 * (base + r1), axis=1, keepdims=True)
    pos2 = jnp.sum(oh2 * (base + r2), axis=1, keepdims=True)
    ones128 = jnp.ones((1, 128), jnp.float32)
    pos1_ref[...] = lax.squeeze(pos1.astype(jnp.int32), [1])
    pos2_ref[...] = lax.squeeze(pos2.astype(jnp.int32), [1])
    w1_ref[...] = (m1 / denom) * ones128
    w2_ref[...] = (m2 / denom) * ones128
    bi = jax.lax.broadcasted_iota(jnp.int32, (32, E), 0)
    be = jnp.sum((bi >= block_start).astype(jnp.int32), axis=1,
                 keepdims=True) - 1
    be_ref[...] = lax.squeeze(jnp.clip(be, 0, E - 1), [1])
    total = jnp.sum(blocks, axis=1, keepdims=True)
    bi1 = jax.lax.broadcasted_iota(jnp.int32, (32, 1), 0)
    act_ref[...] = lax.squeeze((bi1 < total).astype(jnp.int32), [1])

    y = jnp.zeros((N, H), jnp.float32) + sdnb_ref[...]
    for k in range(DFF // DC):
        hk = jax.lax.dot_general(
            x_ref[...], sup_ref[pl.ds(k * DC, DC), :],
            (((1,), (1,)), ((), ())), preferred_element_type=jnp.float32)
        hk = _gelu(hk + supb_ref[:, pl.ds(k * DC, DC)])
        y = y + jax.lax.dot_general(
            hk, sdn_ref[:, pl.ds(k * DC, DC)],
            (((1,), (1,)), ((), ())), preferred_element_type=jnp.float32)
    sh_ref[...] = np.float32(0.1) * y


def _router(xf, gate_W, gb, sup_W, sup_b, sdown_W, sdown_b):
    return pl.pallas_call(
        _router_body,
        out_shape=[
            jax.ShapeDtypeStruct((N,), jnp.int32),
            jax.ShapeDtypeStruct((N,), jnp.int32),
            jax.ShapeDtypeStruct((N, 128), jnp.float32),
            jax.ShapeDtypeStruct((N, 128), jnp.float32),
            jax.ShapeDtypeStruct((32,), jnp.int32),
            jax.ShapeDtypeStruct((32,), jnp.int32),
            jax.ShapeDtypeStruct((N, H), jnp.float32),
        ],
    )(xf, gate_W, gb, sup_W, sup_b[None, :], sdown_W, sdown_b[None, :])


def _dispatch_body(xf_hbm, p1_hbm, p2_hbm, xs_hbm, p1_v, p2_v, rows_v, sem):
    wid = lax.axis_index("s") * 2 + lax.axis_index("c")
    b = wid * TPW
    pltpu.sync_copy(p1_hbm.at[pl.ds(b, TPW)], p1_v)
    pltpu.sync_copy(p2_hbm.at[pl.ds(b, TPW)], p2_v)
    pltpu.sync_copy(xf_hbm.at[pl.ds(b, TPW)], rows_v)
    c1 = pltpu.async_copy(rows_v, xs_hbm.at[p1_v], sem)
    c2 = pltpu.async_copy(rows_v, xs_hbm.at[p2_v], sem)
    c1.wait()
    c2.wait()


def _dispatch(xf, pos1, pos2):
    mesh = plsc.VectorSubcoreMesh(core_axis_name="c", subcore_axis_name="s")
    f = functools.partial(
        pl.kernel, mesh=mesh,
        out_type=jax.ShapeDtypeStruct((P, H), jnp.float32),
        scratch_types=[
            pltpu.VMEM((TPW,), jnp.int32),
            pltpu.VMEM((TPW,), jnp.int32),
            pltpu.VMEM((TPW, H), jnp.float32),
            pltpu.SemaphoreType.DMA,
        ],
    )(_dispatch_body)
    return f(xf, pos1, pos2)


def _grouped_body(be_ref, act_ref, xs_ref, up_ref, upb_ref, dn_ref, dnb_ref,
                  ys_ref):
    i = pl.program_id(0)

    @pl.when(act_ref[i] != 0)
    def _compute():
        h = jax.lax.dot_general(
            xs_ref[...], up_ref[0], (((1,), (1,)), ((), ())),
            preferred_element_type=jnp.float32)
        h = _gelu(h + upb_ref[0])
        y = jax.lax.dot_general(
            h, dn_ref[0], (((1,), (1,)), ((), ())),
            preferred_element_type=jnp.float32)
        ys_ref[...] = y + dnb_ref[0]


def _grouped(be, act, xs, up_W, up_b, down_W, down_b):
    grid_spec = pltpu.PrefetchScalarGridSpec(
        num_scalar_prefetch=2,
        grid=(NBLK,),
        in_specs=[
            pl.BlockSpec((MB, H), lambda i, be, act: (i, 0)),
            pl.BlockSpec((1, DFF, H), lambda i, be, act: (be[i], 0, 0)),
            pl.BlockSpec((1, 1, DFF), lambda i, be, act: (be[i], 0, 0)),
            pl.BlockSpec((1, H, DFF), lambda i, be, act: (be[i], 0, 0)),
            pl.BlockSpec((1, 1, H), lambda i, be, act: (be[i], 0, 0)),
        ],
        out_specs=pl.BlockSpec((MB, H), lambda i, be, act: (i, 0)),
    )
    return pl.pallas_call(
        _grouped_body,
        grid_spec=grid_spec,
        out_shape=jax.ShapeDtypeStruct((P, H), jnp.float32),
        compiler_params=pltpu.CompilerParams(
            dimension_semantics=("arbitrary",)),
    )(be, act, xs, up_W, up_b[:, None, :], down_W, down_b[:, None, :])


def _combine_body(sh_hbm, ys_hbm, p1_hbm, p2_hbm, w1_hbm, w2_hbm, out_hbm,
                  p1_v, p2_v, acc_v, g_v, w_v, sem):
    wid = lax.axis_index("s") * 2 + lax.axis_index("c")
    b = wid * TPW
    pltpu.sync_copy(p1_hbm.at[pl.ds(b, TPW)], p1_v)
    pltpu.sync_copy(p2_hbm.at[pl.ds(b, TPW)], p2_v)
    pltpu.sync_copy(sh_hbm.at[pl.ds(b, TPW)], acc_v)

    nchunk = H // 16

    def add_row(i, _):
        wv = w_v[i, pl.ds(0, 16)]
        for c in range(nchunk):
            sl = pl.ds(c * 16, 16)
            acc_v[i, sl] = acc_v[i, sl] + wv * g_v[i, sl]
        return 0

    pltpu.sync_copy(w1_hbm.at[pl.ds(b, TPW)], w_v)
    pltpu.async_copy(ys_hbm.at[p1_v], g_v, sem).wait()
    lax.fori_loop(0, TPW, add_row, 0)
    pltpu.sync_copy(w2_hbm.at[pl.ds(b, TPW)], w_v)
    pltpu.async_copy(ys_hbm.at[p2_v], g_v, sem).wait()
    lax.fori_loop(0, TPW, add_row, 0)
    pltpu.sync_copy(acc_v, out_hbm.at[pl.ds(b, TPW)])


def _combine(sh, ys, pos1, pos2, w1b, w2b):
    mesh = plsc.VectorSubcoreMesh(core_axis_name="c", subcore_axis_name="s")
    f = functools.partial(
        pl.kernel, mesh=mesh,
        out_type=jax.ShapeDtypeStruct((N, H), jnp.float32),
        scratch_types=[
            pltpu.VMEM((TPW,), jnp.int32),
            pltpu.VMEM((TPW,), jnp.int32),
            pltpu.VMEM((TPW, H), jnp.float32),
            pltpu.VMEM((TPW, H), jnp.float32),
            pltpu.VMEM((TPW, 128), jnp.float32),
            pltpu.SemaphoreType.DMA,
        ],
    )(_combine_body)
    return f(sh, ys, pos1, pos2, w1b, w2b)


def kernel(x, gate_W, gate_bias, up_W, up_b, down_W, down_b, sup_W, sup_b,
           sdown_W, sdown_b):
    b, s, h = x.shape
    xf = x.reshape(-1, h)
    pos1, pos2, w1b, w2b, be, act, sh = _router(
        xf, gate_W, gate_bias[None, :], sup_W, sup_b, sdown_W, sdown_b)
    xs = _dispatch(xf, pos1, pos2)
    ys = _grouped(be, act, xs, up_W, up_b, down_W, down_b)
    out = _combine(sh, ys, pos1, pos2, w1b, w2b)
    return out.reshape(b, s, h)
